# Initial kernel scaffold; baseline (speedup 1.0000x reference)
import jax
import jax.numpy as jnp
from jax.experimental import pallas as pl


def kernel(inputs, edge_index, edge_weight, W_e_ru, b_e_ru, W_e_c, b_e_c, W_d_ru, b_d_ru, W_d_c, b_d_c, W_proj, b_proj):
    raise NotImplementedError("write your pallas kernel here")


# speedup vs baseline: 1.0044x; 1.0044x over previous
import jax
import jax.numpy as jnp
from jax.experimental import pallas as pl

_N = 10000
_B = 4
_SEQ = 12
_HORIZON = 1
_UNITS = 64
_K = 2
_IN_DIM = 1
_OUT_DIM = 1

_TILE = 2000


def _ru_body(x0_ref, x1_ref, x2_ref, w0_ref, w1_ref, w2_ref, b_ref, o_ref):
    acc = jnp.dot(x0_ref[...], w0_ref[...], preferred_element_type=jnp.float32)
    acc += jnp.dot(x1_ref[...], w1_ref[...], preferred_element_type=jnp.float32)
    acc += jnp.dot(x2_ref[...], w2_ref[...], preferred_element_type=jnp.float32)
    o_ref[...] = jax.nn.sigmoid(acc + b_ref[...])


def _c_body(x0_ref, x1_ref, x2_ref, w0_ref, w1_ref, w2_ref, b_ref, u_ref,
            h_ref, o_ref):
    acc = jnp.dot(x0_ref[...], w0_ref[...], preferred_element_type=jnp.float32)
    acc += jnp.dot(x1_ref[...], w1_ref[...], preferred_element_type=jnp.float32)
    acc += jnp.dot(x2_ref[...], w2_ref[...], preferred_element_type=jnp.float32)
    c = jnp.tanh(acc + b_ref[...])
    u = u_ref[...]
    o_ref[...] = u * h_ref[...] + (1.0 - u) * c


def _proj_body(x_ref, w_ref, b_ref, o_ref):
    o_ref[...] = jnp.dot(x_ref[...], w_ref[...],
                         preferred_element_type=jnp.float32) + b_ref[...]


def _gconv_matmul(body, xs, ws, b, extra, out_dim):
    bn, f = xs[0].shape
    grid = bn // _TILE
    x_spec = pl.BlockSpec((_TILE, f), lambda i: (i, 0))
    w_spec = pl.BlockSpec((f, out_dim), lambda i: (0, 0))
    b_spec = pl.BlockSpec((1, out_dim), lambda i: (0, 0))
    e_spec = pl.BlockSpec((_TILE, out_dim), lambda i: (i, 0))
    in_specs = [x_spec] * 3 + [w_spec] * 3 + [b_spec] + [e_spec] * len(extra)
    return pl.pallas_call(
        body,
        grid=(grid,),
        in_specs=in_specs,
        out_specs=pl.BlockSpec((_TILE, out_dim), lambda i: (i, 0)),
        out_shape=jax.ShapeDtypeStruct((bn, out_dim), jnp.float32),
    )(*xs, *ws, b, *extra)


def _dconv(x, src, dst, w):
    msg = x[src] * w[:, None]
    return jax.ops.segment_sum(msg, dst, num_segments=_N)


def _cheb_terms(inp, state, src, dst, w):
    x0c = jnp.concatenate([inp, state], axis=-1)
    bq, nq, fq = x0c.shape
    x0 = jnp.transpose(x0c, (1, 0, 2)).reshape(nq, bq * fq)
    x1 = _dconv(x0, src, dst, w)
    x2 = 2.0 * _dconv(x1, src, dst, w) - x0
    terms = []
    for xm in (x0, x1, x2):
        terms.append(
            jnp.transpose(xm.reshape(nq, bq, fq), (1, 0, 2)).reshape(bq * nq, fq))
    return terms, fq


def _split_w(W, f):
    return [W[i * f:(i + 1) * f] for i in range(3)]


def _dcgru_step(inp, h, W_ru, b_ru, W_c, b_c, src, dst, w):
    bq, nq, _ = inp.shape
    xs, f = _cheb_terms(inp, h, src, dst, w)
    ru = _gconv_matmul(_ru_body, xs, _split_w(W_ru, f), b_ru[None, :], [],
                       2 * _UNITS)
    ru = ru.reshape(bq, nq, 2 * _UNITS)
    r = ru[..., :_UNITS]
    u = ru[..., _UNITS:]
    xs_c, f = _cheb_terms(inp, r * h, src, dst, w)
    u_rows = u.reshape(bq * nq, _UNITS)
    h_rows = h.reshape(bq * nq, _UNITS)
    h_new = _gconv_matmul(_c_body, xs_c, _split_w(W_c, f), b_c[None, :],
                          [u_rows, h_rows], _UNITS)
    return h_new.reshape(bq, nq, _UNITS)


def kernel(inputs, edge_index, edge_weight, W_e_ru, b_e_ru, W_e_c, b_e_c,
           W_d_ru, b_d_ru, W_d_c, b_d_c, W_proj, b_proj):
    src = edge_index[0]
    dst = edge_index[1]
    h = jnp.zeros((_B, _N, _UNITS), jnp.float32)
    for t in range(_SEQ):
        x_t = inputs[t].reshape(_B, _N, _IN_DIM)
        h = _dcgru_step(x_t, h, W_e_ru, b_e_ru, W_e_c, b_e_c, src, dst,
                        edge_weight)
    go = jnp.zeros((_B, _N, _OUT_DIM), jnp.float32)
    dh = h
    outs = []
    for _ in range(_HORIZON):
        dh = _dcgru_step(go, dh, W_d_ru, b_d_ru, W_d_c, b_d_c, src, dst,
                         edge_weight)
        dh_rows = dh.reshape(_B * _N, _UNITS)
        grid = (_B * _N) // _TILE
        proj = pl.pallas_call(
            _proj_body,
            grid=(grid,),
            in_specs=[
                pl.BlockSpec((_TILE, _UNITS), lambda i: (i, 0)),
                pl.BlockSpec((_UNITS, _OUT_DIM), lambda i: (0, 0)),
                pl.BlockSpec((1, _OUT_DIM), lambda i: (0, 0)),
            ],
            out_specs=pl.BlockSpec((_TILE, _OUT_DIM), lambda i: (i, 0)),
            out_shape=jax.ShapeDtypeStruct((_B * _N, _OUT_DIM), jnp.float32),
        )(dh_rows, W_proj, b_proj[None, :])
        outs.append(proj.reshape(_B, _N * _OUT_DIM))
        go = proj.reshape(_B, _N, _OUT_DIM)
    return jnp.stack(outs, 0)
